# baseline (device time: 756825 ns/iter reference)
import jax
import jax.numpy as jnp
from jax import lax
from jax.experimental import pallas as pl
from jax.experimental.pallas import tpu as pltpu

N_EXPERTS = 8
E_LOCAL = 4
C = 640
FC = 256
_VMEM = pltpu.CompilerParams(vmem_limit_bytes=56 * 1024 * 1024)


def _permute_to_slots(x, inv_row, n_slots):
    T, D = x.shape
    R, KC = 640, 1024

    def body(inv_ref, x_ref, o_ref):
        r, k = pl.program_id(0), pl.program_id(1)
        rows = r * R + lax.broadcasted_iota(jnp.int32, (R, KC), 0)
        P = (inv_ref[...] == rows).astype(jnp.bfloat16)
        contrib = jnp.dot(
            P, x_ref[...].astype(jnp.bfloat16),
            preferred_element_type=jnp.float32,
        ).astype(jnp.bfloat16)

        @pl.when(k == 0)
        def _():
            o_ref[...] = contrib

        @pl.when(k != 0)
        def _():
            o_ref[...] += contrib

    return pl.pallas_call(
        body,
        grid=(n_slots // R, T // KC),
        in_specs=[
            pl.BlockSpec((1, KC), lambda r, k: (0, k)),
            pl.BlockSpec((KC, D), lambda r, k: (k, 0)),
        ],
        out_specs=pl.BlockSpec((R, D), lambda r, k: (r, 0)),
        out_shape=jax.ShapeDtypeStruct((n_slots, D), jnp.bfloat16),
        compiler_params=_VMEM,
    )(inv_row, x)


def _gather_from_slots(y1, y2, inv_col, T):
    S = y1.shape[0] + y2.shape[0]
    D = y1.shape[1]
    R, KC = 512, 1280
    nk = S // KC

    def body(inv_ref, y1_ref, y2_ref, o_ref):
        k = pl.program_id(1)
        cols = k * KC + lax.broadcasted_iota(jnp.int32, (R, KC), 1)
        P = (inv_ref[...] == cols).astype(jnp.bfloat16)
        yk = jnp.where(k < nk // 2, y1_ref[...], y2_ref[...])
        contrib = jnp.dot(P, yk, preferred_element_type=jnp.float32)

        @pl.when(k == 0)
        def _():
            o_ref[...] = contrib

        @pl.when(k != 0)
        def _():
            o_ref[...] += contrib

    return pl.pallas_call(
        body,
        grid=(T // R, nk),
        in_specs=[
            pl.BlockSpec((R, 1), lambda r, k: (r, 0)),
            pl.BlockSpec((KC, D), lambda r, k: (jnp.minimum(k, 1), 0)),
            pl.BlockSpec((KC, D), lambda r, k: (jnp.maximum(k - 2, 0), 0)),
        ],
        out_specs=pl.BlockSpec((R, D), lambda r, k: (r, 0)),
        out_shape=jax.ShapeDtypeStruct((T, D), jnp.float32),
        compiler_params=_VMEM,
    )(inv_col, y1, y2)


def _moe_middle(local_buf, send_buf, W1, W2):
    n_e, c, d = local_buf.shape
    f = W1.shape[2]
    nk = f // FC

    def body(l_ref, send_ref, w1_ref, w2_ref, o1_ref, yback_ref,
             recv_buf, yrem_buf, acc_ref,
             send_sems, recv_sems, ret_send_sems, ret_recv_sems):
        e, k = pl.program_id(0), pl.program_id(1)
        my_x = lax.axis_index("x")
        my_y = lax.axis_index("y")
        my_z = lax.axis_index("z")
        partner = (1 - my_x, my_y, my_z)

        def dispatch(ee):
            return pltpu.make_async_remote_copy(
                src_ref=send_ref.at[ee],
                dst_ref=recv_buf.at[ee],
                send_sem=send_sems.at[ee],
                recv_sem=recv_sems.at[ee],
                device_id=partner,
                device_id_type=pl.DeviceIdType.MESH,
            )

        def ret(ee):
            return pltpu.make_async_remote_copy(
                src_ref=yrem_buf.at[ee],
                dst_ref=yback_ref.at[ee],
                send_sem=ret_send_sems.at[ee],
                recv_sem=ret_recv_sems.at[ee],
                device_id=partner,
                device_id_type=pl.DeviceIdType.MESH,
            )

        @pl.when((e == 0) & (k == 0))
        def _():
            barrier = pltpu.get_barrier_semaphore()
            pl.semaphore_signal(
                barrier, inc=1, device_id=partner,
                device_id_type=pl.DeviceIdType.MESH,
            )
            pl.semaphore_wait(barrier, 1)
            for ee in range(n_e):
                dispatch(ee).start()

        for ee in range(n_e):
            @pl.when((e == ee) & (k == 0))
            def _():
                dispatch(ee).wait_recv()

        xb = jnp.concatenate([l_ref[0], recv_buf[e]], axis=0)
        h = jnp.maximum(
            jnp.dot(
                xb, w1_ref[0].astype(jnp.bfloat16),
                preferred_element_type=jnp.float32,
            ),
            0.0,
        ).astype(jnp.bfloat16)
        contrib = jnp.dot(
            h, w2_ref[0].astype(jnp.bfloat16),
            preferred_element_type=jnp.float32,
        )

        @pl.when(k == 0)
        def _():
            acc_ref[...] = contrib

        @pl.when(k != 0)
        def _():
            acc_ref[...] += contrib

        for ee in range(n_e):
            @pl.when((e == ee) & (k == nk - 1))
            def _():
                o1_ref[0] = acc_ref[:c].astype(jnp.bfloat16)
                yrem_buf[ee] = acc_ref[c:].astype(jnp.bfloat16)
                ret(ee).start()

        @pl.when((e == n_e - 1) & (k == nk - 1))
        def _():
            for ee in range(n_e):
                dispatch(ee).wait_send()
                ret(ee).wait_send()
                ret(ee).wait_recv()

    return pl.pallas_call(
        body,
        grid=(n_e, nk),
        in_specs=[
            pl.BlockSpec((1, c, d), lambda e, k: (e, 0, 0)),
            pl.BlockSpec(memory_space=pl.ANY),
            pl.BlockSpec((1, d, FC), lambda e, k: (e, 0, k)),
            pl.BlockSpec((1, FC, d), lambda e, k: (e, k, 0)),
        ],
        out_specs=[
            pl.BlockSpec((1, c, d), lambda e, k: (e, 0, 0)),
            pl.BlockSpec(memory_space=pl.ANY),
        ],
        out_shape=[
            jax.ShapeDtypeStruct((n_e, c, d), jnp.bfloat16),
            jax.ShapeDtypeStruct((n_e, c, d), jnp.bfloat16),
        ],
        scratch_shapes=[
            pltpu.VMEM((n_e, c, d), jnp.bfloat16),
            pltpu.VMEM((n_e, c, d), jnp.bfloat16),
            pltpu.VMEM((2 * c, d), jnp.float32),
            pltpu.SemaphoreType.DMA((n_e,)),
            pltpu.SemaphoreType.DMA((n_e,)),
            pltpu.SemaphoreType.DMA((n_e,)),
            pltpu.SemaphoreType.DMA((n_e,)),
        ],
        compiler_params=pltpu.CompilerParams(
            collective_id=0,
            vmem_limit_bytes=60 * 1024 * 1024,
        ),
    )(local_buf, send_buf, W1, W2)


def kernel(x, assign, W1, W2):
    T, D = x.shape
    my_x = lax.axis_index("x")
    my_base = my_x * E_LOCAL

    oh = (assign[:, None] == jnp.arange(N_EXPERTS)[None, :]).astype(jnp.int32)
    rank = (oh * (jnp.cumsum(oh, axis=0) - oh)).sum(axis=1)
    slot_e = jnp.remainder(assign - my_base, N_EXPERTS)
    inv = (slot_e * C + rank).astype(jnp.int32)

    nloc = E_LOCAL * C
    gathered = _permute_to_slots(x, inv.reshape(1, T), 2 * nloc)
    local_buf = gathered[:nloc].reshape(E_LOCAL, C, D)
    send_buf = gathered[nloc:].reshape(E_LOCAL, C, D)

    y_local, y_back = _moe_middle(local_buf, send_buf, W1, W2)

    return _gather_from_slots(
        y_local.reshape(-1, D), y_back.reshape(-1, D), inv.reshape(T, 1), T
    )
